# baseline (speedup 1.0000x reference)
import functools

import jax
import jax.numpy as jnp
from jax import lax
from jax.experimental import pallas as pl
from jax.experimental.pallas import tpu as pltpu
from jax.experimental.pallas import tpu_sc as plsc

_NUM_CATEGORIES = 1000
_BATCH = 16384
_TABLE_PAD = 1024

_NC, _NS, _L = 2, 16, 16
_NW = _NC * _NS
_B_PER_W = _BATCH // _NW


_CHUNK = 128
_N_CHUNK = _B_PER_W // _CHUNK


def _lookup(idx_hbm, tab_hbm, out_hbm, idx_v, out_v, sem):
    wid = lax.axis_index("s") * _NC + lax.axis_index("c")
    base = wid * _B_PER_W
    pltpu.sync_copy(idx_hbm.at[pl.ds(wid * _N_CHUNK, _N_CHUNK)], idx_v)
    copies = [
        pltpu.async_copy(
            tab_hbm.at[idx_v.at[j]], out_v.at[pl.ds(j * _CHUNK, _CHUNK)], sem
        )
        for j in range(_N_CHUNK)
    ]
    for c in copies:
        c.wait()
    pltpu.sync_copy(out_v, out_hbm.at[pl.ds(base, _B_PER_W)])


_sc_call = functools.partial(
    pl.kernel,
    out_type=jax.ShapeDtypeStruct((_BATCH,), jnp.float32),
    mesh=plsc.VectorSubcoreMesh(core_axis_name="c", subcore_axis_name="s"),
    compiler_params=pltpu.CompilerParams(needs_layout_passes=False),
    scratch_types=[
        pltpu.VMEM((_N_CHUNK, _CHUNK), jnp.int32),
        pltpu.VMEM((_B_PER_W,), jnp.float32),
        pltpu.SemaphoreType.DMA,
    ],
)(_lookup)


@jax.jit
def kernel(category_index, W):
    idx = category_index.astype(jnp.int32).reshape(_NW * _N_CHUNK, _CHUNK)
    return _sc_call(idx, W.reshape(-1))


# speedup vs baseline: 1.5256x; 1.5256x over previous
import functools

import jax
import jax.numpy as jnp
from jax import lax
from jax.experimental import pallas as pl
from jax.experimental.pallas import tpu as pltpu
from jax.experimental.pallas import tpu_sc as plsc

_NUM_CATEGORIES = 1000
_BATCH = 16384
_TABLE_PAD = 1024

_NC, _NS, _L = 1, 16, 16
_NW = _NC * _NS
_B_PER_W = _BATCH // _NW


def _lookup(idx_hbm, tab_hbm, out_hbm, idx_v, tab_v, out_v, sem_t, sem_i):
    wid = lax.axis_index("s") * _NC + lax.axis_index("c")
    base = wid * _B_PER_W
    ct = pltpu.async_copy(tab_hbm, tab_v, sem_t)
    ci = pltpu.async_copy(idx_hbm.at[pl.ds(base, _B_PER_W)], idx_v, sem_i)
    ct.wait()
    ci.wait()
    for i in range(_B_PER_W // _L):
        iv = idx_v[pl.ds(i * _L, _L)]
        out_v[pl.ds(i * _L, _L)] = plsc.load_gather(tab_v, [iv])
    pltpu.sync_copy(out_v, out_hbm.at[pl.ds(base, _B_PER_W)])


_sc_call = functools.partial(
    pl.kernel,
    out_type=jax.ShapeDtypeStruct((_BATCH,), jnp.float32),
    mesh=plsc.VectorSubcoreMesh(
        core_axis_name="c", subcore_axis_name="s", num_cores=_NC
    ),
    compiler_params=pltpu.CompilerParams(needs_layout_passes=False),
    scratch_types=[
        pltpu.VMEM((_B_PER_W,), jnp.int32),
        pltpu.VMEM((_NUM_CATEGORIES,), jnp.float32),
        pltpu.VMEM((_B_PER_W,), jnp.float32),
        pltpu.SemaphoreType.DMA,
        pltpu.SemaphoreType.DMA,
    ],
)(_lookup)


@jax.jit
def kernel(category_index, W):
    idx = category_index.astype(jnp.int32)
    return _sc_call(idx, W.reshape(-1))
